# baseline (device time: 140365 ns/iter reference)
import functools

import jax
import jax.numpy as jnp
from jax import lax
from jax.experimental import pallas as pl
from jax.experimental.pallas import tpu as pltpu

N_DEV = 4
HC = 512


def _mlp_partial_body(x_ref, win_ref, wout_ref, out_ref):
    c = pl.program_id(0)
    xb = x_ref[...].astype(jnp.bfloat16)
    wi = win_ref[...].astype(jnp.bfloat16)
    h = lax.dot(xb, wi, preferred_element_type=jnp.float32)
    hb = jnp.maximum(h, 0.0).astype(jnp.bfloat16)
    wo = wout_ref[...].astype(jnp.bfloat16)
    p = lax.dot(hb, wo, preferred_element_type=jnp.float32)

    @pl.when(c == 0)
    def _():
        out_ref[...] = p

    @pl.when(c != 0)
    def _():
        out_ref[...] = out_ref[...] + p


def _mlp_partial(x, win, wout):
    b, d = x.shape
    _, h_sh = win.shape
    assert h_sh % HC == 0
    return pl.pallas_call(
        _mlp_partial_body,
        grid=(h_sh // HC,),
        in_specs=[
            pl.BlockSpec((b, d), lambda c: (0, 0)),
            pl.BlockSpec((d, HC), lambda c: (0, c)),
            pl.BlockSpec((HC, d), lambda c: (c, 0)),
        ],
        out_specs=pl.BlockSpec((b, d), lambda c: (0, 0)),
        out_shape=jax.ShapeDtypeStruct((b, d), jnp.float32),
    )(x, win, wout)


def _allreduce_body(scatter, x_ref, out_ref, comm_ref, send_sems, recv_sems):
    my = lax.axis_index("i")
    left = (my + N_DEV - 1) % N_DEV
    right = (my + 1) % N_DEV

    barrier = pltpu.get_barrier_semaphore()
    for nbr in (left, right):
        pl.semaphore_signal(
            barrier, inc=1, device_id=(nbr,), device_id_type=pl.DeviceIdType.MESH
        )
    pl.semaphore_wait(barrier, 2)

    comm_ref[0] = x_ref[...]
    for hop in range(N_DEV - 1):
        rdma = pltpu.make_async_remote_copy(
            src_ref=comm_ref.at[hop],
            dst_ref=comm_ref.at[hop + 1],
            send_sem=send_sems.at[hop],
            recv_sem=recv_sems.at[hop + 1],
            device_id=(right,),
            device_id_type=pl.DeviceIdType.MESH,
        )
        rdma.start()
        rdma.wait()

    if scatter:
        rows = out_ref.shape[0]
        start = my * rows
        out_ref[...] = (
            comm_ref[0, pl.ds(start, rows), :]
            + comm_ref[1, pl.ds(start, rows), :]
            + comm_ref[2, pl.ds(start, rows), :]
            + comm_ref[3, pl.ds(start, rows), :]
        )
    else:
        out_ref[...] = comm_ref[0] + comm_ref[1] + comm_ref[2] + comm_ref[3]


def _allreduce(x, collective_id, scatter=False):
    b, d = x.shape
    out_rows = b // N_DEV if scatter else b
    return pl.pallas_call(
        functools.partial(_allreduce_body, scatter),
        out_shape=jax.ShapeDtypeStruct((out_rows, d), x.dtype),
        in_specs=[pl.BlockSpec(memory_space=pltpu.VMEM)],
        out_specs=pl.BlockSpec(memory_space=pltpu.VMEM),
        scratch_shapes=[
            pltpu.VMEM((N_DEV, b, d), x.dtype),
            pltpu.SemaphoreType.DMA((N_DEV,)),
            pltpu.SemaphoreType.DMA((N_DEV,)),
        ],
        compiler_params=pltpu.CompilerParams(collective_id=collective_id),
    )(x)


def kernel(x, Win0, Wout0, Win1, Wout1, Win2, Wout2):
    p0 = _mlp_partial(x, Win0, Wout0)
    x1 = _allreduce(p0, collective_id=0)
    p1 = _mlp_partial(x1, Win1, Wout1)
    x2 = _allreduce(p1, collective_id=1)
    p2 = _mlp_partial(x2, Win2, Wout2)
    return _allreduce(p2, collective_id=2, scatter=True)


# device time: 96870 ns/iter; 1.4490x vs baseline; 1.4490x over previous
import functools

import jax
import jax.numpy as jnp
from jax import lax
from jax.experimental import pallas as pl
from jax.experimental.pallas import tpu as pltpu

N_DEV = 4
HC = 512


def _layer_body(
    nsteps,
    ar_in,
    rs_out,
    p_ref,
    win_ref,
    wout_ref,
    out_ref,
    xb_ref,
    acc_ref,
    r1_ref,
    s1_ref,
    r2_ref,
    sb1_ref,
    rb1_ref,
    sb2_ref,
    rb2_ref,
    sh_ref,
    ssems,
    rsems,
):
    c = pl.program_id(0)
    my = lax.axis_index("i")
    p_near = my ^ 1
    p_far = my ^ 2

    b = acc_ref.shape[0]
    half = b // 2
    quarter = b // 4

    if ar_in or rs_out:

        @pl.when(c == 0)
        def _():
            barrier = pltpu.get_barrier_semaphore()
            for nbr in (p_near, p_far):
                pl.semaphore_signal(
                    barrier,
                    inc=1,
                    device_id=(nbr,),
                    device_id_type=pl.DeviceIdType.MESH,
                )
            pl.semaphore_wait(barrier, 2)

    @pl.when(c == 0)
    def _():
        if ar_in:
            rd1 = pltpu.make_async_remote_copy(
                src_ref=p_ref,
                dst_ref=r1_ref,
                send_sem=ssems.at[0],
                recv_sem=rsems.at[0],
                device_id=(p_near,),
                device_id_type=pl.DeviceIdType.MESH,
            )
            rd1.start()
            rd1.wait()
            acc1 = p_ref[...].astype(jnp.float32) + r1_ref[...].astype(jnp.float32)
            s1_ref[...] = acc1.astype(jnp.bfloat16)
            rd2 = pltpu.make_async_remote_copy(
                src_ref=s1_ref,
                dst_ref=r2_ref,
                send_sem=ssems.at[1],
                recv_sem=rsems.at[1],
                device_id=(p_far,),
                device_id_type=pl.DeviceIdType.MESH,
            )
            rd2.start()
            rd2.wait()
            xb_ref[...] = (acc1 + r2_ref[...].astype(jnp.float32)).astype(
                jnp.bfloat16
            )
        else:
            xb_ref[...] = p_ref[...].astype(jnp.bfloat16)

    wi = win_ref[...].astype(jnp.bfloat16)
    h = lax.dot(xb_ref[...], wi, preferred_element_type=jnp.float32)
    hb = jnp.maximum(h, 0.0).astype(jnp.bfloat16)
    wo = wout_ref[...].astype(jnp.bfloat16)
    p = lax.dot(hb, wo, preferred_element_type=jnp.float32)

    @pl.when(c == 0)
    def _():
        acc_ref[...] = p

    @pl.when(c != 0)
    def _():
        acc_ref[...] = acc_ref[...] + p

    @pl.when(c == nsteps - 1)
    def _():
        if rs_out:
            my_half = (my // 2) * half
            far_half = half - my_half
            sb1_ref[...] = acc_ref[pl.ds(far_half, half), :].astype(jnp.bfloat16)
            rr1 = pltpu.make_async_remote_copy(
                src_ref=sb1_ref,
                dst_ref=rb1_ref,
                send_sem=ssems.at[2],
                recv_sem=rsems.at[2],
                device_id=(p_far,),
                device_id_type=pl.DeviceIdType.MESH,
            )
            rr1.start()
            rr1.wait()
            sh_ref[...] = acc_ref[pl.ds(my_half, half), :] + rb1_ref[...].astype(
                jnp.float32
            )
            near_q = (1 - my % 2) * quarter
            my_q = (my % 2) * quarter
            sb2_ref[...] = sh_ref[pl.ds(near_q, quarter), :].astype(jnp.bfloat16)
            rr2 = pltpu.make_async_remote_copy(
                src_ref=sb2_ref,
                dst_ref=rb2_ref,
                send_sem=ssems.at[3],
                recv_sem=rsems.at[3],
                device_id=(p_near,),
                device_id_type=pl.DeviceIdType.MESH,
            )
            rr2.start()
            rr2.wait()
            out_ref[...] = sh_ref[pl.ds(my_q, quarter), :] + rb2_ref[...].astype(
                jnp.float32
            )
        else:
            out_ref[...] = acc_ref[...].astype(jnp.bfloat16)


def _layer(p, win, wout, *, collective_id=None, ar_in=False, rs_out=False):
    b = p.shape[0]
    d = win.shape[0]
    h_sh = win.shape[1]
    assert h_sh % HC == 0
    nsteps = h_sh // HC
    if rs_out:
        out_shape = jax.ShapeDtypeStruct((b // N_DEV, d), jnp.float32)
    else:
        out_shape = jax.ShapeDtypeStruct((b, d), jnp.bfloat16)
    params = {}
    if collective_id is not None:
        params["compiler_params"] = pltpu.CompilerParams(
            collective_id=collective_id, dimension_semantics=("arbitrary",)
        )
    else:
        params["compiler_params"] = pltpu.CompilerParams(
            dimension_semantics=("arbitrary",)
        )
    return pl.pallas_call(
        functools.partial(_layer_body, nsteps, ar_in, rs_out),
        grid=(nsteps,),
        in_specs=[
            pl.BlockSpec((b, d), lambda c: (0, 0)),
            pl.BlockSpec((d, HC), lambda c: (0, c)),
            pl.BlockSpec((HC, d), lambda c: (c, 0)),
        ],
        out_specs=pl.BlockSpec(out_shape.shape, lambda c: (0, 0)),
        out_shape=out_shape,
        scratch_shapes=[
            pltpu.VMEM((b, d), jnp.bfloat16),
            pltpu.VMEM((b, d), jnp.float32),
            pltpu.VMEM((b, d), jnp.bfloat16),
            pltpu.VMEM((b, d), jnp.bfloat16),
            pltpu.VMEM((b, d), jnp.bfloat16),
            pltpu.VMEM((b // 2, d), jnp.bfloat16),
            pltpu.VMEM((b // 2, d), jnp.bfloat16),
            pltpu.VMEM((b // 4, d), jnp.bfloat16),
            pltpu.VMEM((b // 4, d), jnp.bfloat16),
            pltpu.VMEM((b // 2, d), jnp.float32),
            pltpu.SemaphoreType.DMA((4,)),
            pltpu.SemaphoreType.DMA((4,)),
        ],
        **params,
    )(p, win, wout)


def kernel(x, Win0, Wout0, Win1, Wout1, Win2, Wout2):
    p0 = _layer(x, Win0, Wout0)
    p1 = _layer(p0, Win1, Wout1, collective_id=0, ar_in=True)
    return _layer(p1, Win2, Wout2, collective_id=1, ar_in=True, rs_out=True)


# device time: 93090 ns/iter; 1.5078x vs baseline; 1.0406x over previous
import functools

import jax
import jax.numpy as jnp
from jax import lax
from jax.experimental import pallas as pl
from jax.experimental.pallas import tpu as pltpu

N_DEV = 4
NH = 8
NJ = 4
XOR = (1, 3, 2)


def _layer_body(
    in_f32,
    rs_out,
    x_ref,
    win_ref,
    wout_ref,
    out_ref,
    xb_ref,
    h_ref,
    pbuf_ref,
    pslab_ref,
    rbuf_ref,
    rrbuf_ref,
    ssems,
    rsems,
    rs_ssems,
    rs_rsems,
):
    c = pl.program_id(0)
    my = lax.axis_index("i")
    b = h_ref.shape[0]
    d = out_ref.shape[1]
    hc = d // NJ
    rows = b // N_DEV

    @pl.when(c == 0)
    def _():
        barrier = pltpu.get_barrier_semaphore()
        for m in XOR:
            pl.semaphore_signal(
                barrier,
                inc=1,
                device_id=(my ^ m,),
                device_id_type=pl.DeviceIdType.MESH,
            )
        pl.semaphore_wait(barrier, 3)
        if in_f32:
            xb_ref[...] = x_ref[...].astype(jnp.bfloat16)

    xsrc = xb_ref if in_f32 else x_ref

    @pl.when(c < NH)
    def _():
        wi = win_ref[...].astype(jnp.bfloat16)
        hv = lax.dot(xsrc[...], wi, preferred_element_type=jnp.float32)
        h_ref[:, pl.ds(c * (h_ref.shape[1] // NH), h_ref.shape[1] // NH)] = (
            jnp.maximum(hv, 0.0).astype(jnp.bfloat16)
        )

    @pl.when(c >= NH)
    def _():
        j = c - NH
        wo = wout_ref[...].astype(jnp.bfloat16)
        pv = lax.dot(h_ref[...], wo, preferred_element_type=jnp.float32)
        pvb = pv.astype(jnp.bfloat16)
        if rs_out:
            pslab_ref[:, pl.ds(j * hc, hc)] = pvb
        else:
            pbuf_ref[j] = pvb
            for k, m in enumerate(XOR):
                rdma = pltpu.make_async_remote_copy(
                    src_ref=pbuf_ref.at[j],
                    dst_ref=rbuf_ref.at[j, k],
                    send_sem=ssems.at[j * 3 + k],
                    recv_sem=rsems.at[j * 3 + k],
                    device_id=(my ^ m,),
                    device_id_type=pl.DeviceIdType.MESH,
                )
                rdma.start()

    @pl.when(c == NH + NJ - 1)
    def _():
        if rs_out:
            for k, m in enumerate(XOR):
                peer = my ^ m
                rdma = pltpu.make_async_remote_copy(
                    src_ref=pslab_ref.at[pl.ds(peer * rows, rows), :],
                    dst_ref=rrbuf_ref.at[k],
                    send_sem=rs_ssems.at[k],
                    recv_sem=rs_rsems.at[k],
                    device_id=(peer,),
                    device_id_type=pl.DeviceIdType.MESH,
                )
                rdma.start()
            for k in range(3):
                pltpu.make_async_remote_copy(
                    src_ref=pslab_ref.at[pl.ds(0, rows), :],
                    dst_ref=rrbuf_ref.at[k],
                    send_sem=rs_ssems.at[k],
                    recv_sem=rs_rsems.at[k],
                    device_id=(my,),
                    device_id_type=pl.DeviceIdType.MESH,
                ).wait()
            out_ref[...] = (
                pslab_ref[pl.ds(my * rows, rows), :].astype(jnp.float32)
                + rrbuf_ref[0].astype(jnp.float32)
                + rrbuf_ref[1].astype(jnp.float32)
                + rrbuf_ref[2].astype(jnp.float32)
            )
        else:
            for j in range(NJ):
                for k in range(3):
                    pltpu.make_async_remote_copy(
                        src_ref=pbuf_ref.at[j],
                        dst_ref=rbuf_ref.at[j, k],
                        send_sem=ssems.at[j * 3 + k],
                        recv_sem=rsems.at[j * 3 + k],
                        device_id=(my,),
                        device_id_type=pl.DeviceIdType.MESH,
                    ).wait()
                out_ref[:, pl.ds(j * hc, hc)] = (
                    pbuf_ref[j].astype(jnp.float32)
                    + rbuf_ref[j, 0].astype(jnp.float32)
                    + rbuf_ref[j, 1].astype(jnp.float32)
                    + rbuf_ref[j, 2].astype(jnp.float32)
                ).astype(jnp.bfloat16)


def _layer(x, win, wout, *, collective_id, in_f32, rs_out):
    b = x.shape[0]
    d = win.shape[0]
    h_sh = win.shape[1]
    hcw = h_sh // NH
    hc = d // NJ
    rows = b // N_DEV
    if rs_out:
        out_shape = jax.ShapeDtypeStruct((rows, d), jnp.float32)
    else:
        out_shape = jax.ShapeDtypeStruct((b, d), jnp.bfloat16)
    return pl.pallas_call(
        functools.partial(_layer_body, in_f32, rs_out),
        grid=(NH + NJ,),
        in_specs=[
            pl.BlockSpec((b, d), lambda c: (0, 0)),
            pl.BlockSpec((d, hcw), lambda c: (0, jnp.minimum(c, NH - 1))),
            pl.BlockSpec((h_sh, hc), lambda c: (0, jnp.maximum(c - NH, 0))),
        ],
        out_specs=pl.BlockSpec(out_shape.shape, lambda c: (0, 0)),
        out_shape=out_shape,
        scratch_shapes=[
            pltpu.VMEM((b, d), jnp.bfloat16),
            pltpu.VMEM((b, h_sh), jnp.bfloat16),
            pltpu.VMEM((NJ, b, hc), jnp.bfloat16),
            pltpu.VMEM((b, d), jnp.bfloat16),
            pltpu.VMEM((NJ, 3, b, hc), jnp.bfloat16),
            pltpu.VMEM((3, rows, d), jnp.bfloat16),
            pltpu.SemaphoreType.DMA((NJ * 3,)),
            pltpu.SemaphoreType.DMA((NJ * 3,)),
            pltpu.SemaphoreType.DMA((3,)),
            pltpu.SemaphoreType.DMA((3,)),
        ],
        compiler_params=pltpu.CompilerParams(
            collective_id=collective_id, dimension_semantics=("arbitrary",)
        ),
    )(x, win, wout)


def kernel(x, Win0, Wout0, Win1, Wout1, Win2, Wout2):
    x1 = _layer(x, Win0, Wout0, collective_id=0, in_f32=True, rs_out=False)
    x2 = _layer(x1, Win1, Wout1, collective_id=1, in_f32=False, rs_out=False)
    return _layer(x2, Win2, Wout2, collective_id=2, in_f32=False, rs_out=True)
